# Initial kernel scaffold; baseline (speedup 1.0000x reference)
import jax
import jax.numpy as jnp
from jax.experimental import pallas as pl


def kernel(x, edge_index, batch, W1_rel, W1_root, b1, W2_rel, W2_root, b2, W3_rel, W3_root, b3, Wlin, blin):
    raise NotImplementedError("write your pallas kernel here")


# speedup vs baseline: 25.6464x; 25.6464x over previous
import functools

import jax
import jax.numpy as jnp
from jax import lax
from jax.experimental import pallas as pl
from jax.experimental.pallas import tpu as pltpu
from jax.experimental.pallas import tpu_sc as plsc

N = 100000
NP = 100352
E = 3200000
EP = 3276800
G = 64

NW = 32
EROWS = EP // 128
ROWS_PER_W = EROWS // NW
KB = 8
NBLK = ROWS_PER_W // KB
TROWS = NP // 16


def _sc_segsum(hpre, srcp, dstp, zeros, w):
    mesh = plsc.VectorSubcoreMesh(core_axis_name="c", subcore_axis_name="s")

    @functools.partial(
        pl.kernel,
        out_type=jax.ShapeDtypeStruct((2 * NP, w), jnp.float32),
        mesh=mesh,
        scratch_types=[
            pltpu.VMEM((KB, 128), jnp.int32),
            pltpu.VMEM((KB, 128), jnp.int32),
            pltpu.VMEM((KB, 128, w), jnp.float32),
            pltpu.VMEM_SHARED((NP, w), jnp.float32),
            pltpu.SemaphoreType.DMA,
            pltpu.SemaphoreType.DMA,
        ],
        compiler_params=pltpu.CompilerParams(use_tc_tiling_on_sc=False),
    )
    def k(hpre_h, src_h, dst_h, zeros_h, out_h, srcv, dstv, rows, acc, gsem, ssem):
        cid = lax.axis_index("c")
        sid = lax.axis_index("s")
        pltpu.sync_copy(zeros_h, acc.at[pl.ds(sid * TROWS, TROWS)])
        plsc.subcore_barrier()
        wid = sid * 2 + cid
        row0 = wid * ROWS_PER_W

        def blk(b, carry):
            r0 = row0 + b * KB
            pltpu.sync_copy(src_h.at[pl.ds(r0, KB)], srcv)
            pltpu.sync_copy(dst_h.at[pl.ds(r0, KB)], dstv)
            gds = [pltpu.async_copy(hpre_h.at[srcv.at[j]], rows.at[j], gsem)
                   for j in range(KB)]
            for d in gds:
                d.wait()
            sds = [pltpu.async_copy(rows.at[j], acc.at[dstv.at[j]], ssem, add=True)
                   for j in range(KB)]
            for d in sds:
                d.wait()
            return carry

        lax.fori_loop(0, NBLK, blk, 0)
        plsc.subcore_barrier()
        pltpu.sync_copy(acc.at[pl.ds(sid * TROWS, TROWS)],
                        out_h.at[pl.ds(cid * NP + sid * TROWS, TROWS)])

    return k(hpre, srcp, dstp, zeros)


def _tc1(P1p, x8p, BD1rel, BD1root, BD2rel, BD2root, b2t):
    RB = 896

    def body(p1_ref, x_ref, br_ref, bo_ref, c2r_ref, c2o_ref, b2_ref,
             p2_ref, q2_ref):
        agg = p1_ref[0] + p1_ref[1]
        h1 = jnp.maximum(
            jnp.dot(agg, br_ref[...], preferred_element_type=jnp.float32, precision=lax.Precision.HIGHEST)
            + jnp.dot(x_ref[...], bo_ref[...], preferred_element_type=jnp.float32, precision=lax.Precision.HIGHEST),
            0.0)
        del b2_ref
        p2_ref[...] = jnp.dot(h1, c2r_ref[...], preferred_element_type=jnp.float32, precision=lax.Precision.HIGHEST)
        q2_ref[...] = jnp.dot(h1, c2o_ref[...], preferred_element_type=jnp.float32, precision=lax.Precision.HIGHEST)

    return pl.pallas_call(
        body,
        grid=(6272 // RB,),
        in_specs=[
            pl.BlockSpec((2, RB, 128), lambda i: (0, i, 0)),
            pl.BlockSpec((RB, 128), lambda i: (i, 0)),
            pl.BlockSpec((128, 512), lambda i: (0, 0)),
            pl.BlockSpec((128, 512), lambda i: (0, 0)),
            pl.BlockSpec((512, 256), lambda i: (0, 0)),
            pl.BlockSpec((512, 256), lambda i: (0, 0)),
            pl.BlockSpec((1, 256), lambda i: (0, 0)),
        ],
        out_specs=[
            pl.BlockSpec((RB, 256), lambda i: (i, 0)),
            pl.BlockSpec((RB, 256), lambda i: (i, 0)),
        ],
        out_shape=[
            jax.ShapeDtypeStruct((6272, 256), jnp.float32),
            jax.ShapeDtypeStruct((6272, 256), jnp.float32),
        ],
        compiler_params=pltpu.CompilerParams(
            dimension_semantics=("parallel",)),
    )(P1p, x8p, BD1rel, BD1root, BD2rel, BD2root, b2t)


def _tc2(P2v, q2v, BD3rel, BD3root, b2t256, b3t):
    RB = 896

    def body(p2_ref, q2_ref, c3r_ref, c3o_ref, b2_ref, b3_ref, p3_ref, q3_ref):
        h2 = jnp.maximum(p2_ref[0] + p2_ref[1] + q2_ref[...] + b2_ref[...], 0.0)
        p3_ref[...] = jnp.dot(h2, c3r_ref[...], preferred_element_type=jnp.float32, precision=lax.Precision.HIGHEST)
        q3_ref[...] = (jnp.dot(h2, c3o_ref[...], preferred_element_type=jnp.float32, precision=lax.Precision.HIGHEST)
                       + b3_ref[...])

    return pl.pallas_call(
        body,
        grid=(6272 // RB,),
        in_specs=[
            pl.BlockSpec((2, RB, 256), lambda i: (0, i, 0)),
            pl.BlockSpec((RB, 256), lambda i: (i, 0)),
            pl.BlockSpec((256, 128), lambda i: (0, 0)),
            pl.BlockSpec((256, 128), lambda i: (0, 0)),
            pl.BlockSpec((1, 256), lambda i: (0, 0)),
            pl.BlockSpec((1, 128), lambda i: (0, 0)),
        ],
        out_specs=[
            pl.BlockSpec((RB, 128), lambda i: (i, 0)),
            pl.BlockSpec((RB, 128), lambda i: (i, 0)),
        ],
        out_shape=[
            jax.ShapeDtypeStruct((6272, 128), jnp.float32),
            jax.ShapeDtypeStruct((6272, 128), jnp.float32),
        ],
        compiler_params=pltpu.CompilerParams(
            dimension_semantics=("parallel",)),
    )(P2v, q2v, BD3rel, BD3root, b2t256, b3t)


def _tc3(P3v, q3v, batchp):
    def body(p3_ref, q3_ref, b_ref, out_ref, h3s):
        g = pl.program_id(0)

        @pl.when(g == 0)
        def _():
            h3s[...] = jnp.maximum(p3_ref[0] + p3_ref[1] + q3_ref[...], 0.0)

        v = jnp.where(b_ref[...] == g, h3s[...], 0.0)
        out_ref[0, ...] = jnp.max(v, axis=0, keepdims=True)

    return pl.pallas_call(
        body,
        grid=(G,),
        in_specs=[
            pl.BlockSpec((2, 6272, 128), lambda g: (0, 0, 0)),
            pl.BlockSpec((6272, 128), lambda g: (0, 0)),
            pl.BlockSpec((6272, 128), lambda g: (0, 0)),
        ],
        out_specs=pl.BlockSpec((1, 1, 128), lambda g: (g, 0, 0)),
        out_shape=jax.ShapeDtypeStruct((G, 1, 128), jnp.float32),
        scratch_shapes=[pltpu.VMEM((6272, 128), jnp.float32)],
        compiler_params=pltpu.CompilerParams(
            dimension_semantics=("arbitrary",),
            vmem_limit_bytes=100 * 1024 * 1024),
    )(P3v, q3v, batchp)


def _tc4(pool128, Wlin128, blin128):
    def body(p_ref, wl_ref, bl_ref, out_ref):
        lanes = lax.broadcasted_iota(jnp.int32, (G, 128), 1)
        p = p_ref[...]
        cols = [jnp.max(jnp.where(lanes % 8 == f, p, 0.0), axis=1, keepdims=True)
                for f in range(8)]
        pool8 = jnp.concatenate(cols, axis=1)
        out_ref[...] = (jnp.dot(pool8, wl_ref[...],
                                preferred_element_type=jnp.float32, precision=lax.Precision.HIGHEST)
                        + bl_ref[...])

    return pl.pallas_call(
        body,
        out_shape=jax.ShapeDtypeStruct((G, 128), jnp.float32),
    )(pool128, Wlin128, blin128)


def kernel(x, edge_index, batch, W1_rel, W1_root, b1, W2_rel, W2_root, b2,
           W3_rel, W3_root, b3, Wlin, blin):
    f32 = jnp.float32
    src = edge_index[0]
    dst = edge_index[1]
    pad_e = EP - E
    srcp = jnp.concatenate([src, jnp.full((pad_e,), N, jnp.int32)]).reshape(EROWS, 128)
    dstp = jnp.concatenate([dst, jnp.full((pad_e,), N, jnp.int32)]).reshape(EROWS, 128)

    x8 = jnp.concatenate([x, jnp.ones((N, 1), f32), jnp.zeros((N, 4), f32)],
                         axis=1)
    x8 = jnp.pad(x8, ((0, NP - N), (0, 0)))

    I16 = jnp.eye(16, dtype=f32)
    W1r8 = jnp.concatenate([W1_rel, jnp.zeros((5, 32), f32)], axis=0)
    W1o8 = jnp.concatenate([W1_root, b1[None, :], jnp.zeros((4, 32), f32)],
                           axis=0)
    BD1rel = jnp.kron(I16, W1r8)
    BD1root = jnp.kron(I16, W1o8)
    BD2rel = jnp.kron(I16, W2_rel)
    BD2root = jnp.kron(I16, W2_root)
    BD3rel = jnp.kron(I16, W3_rel)
    BD3root = jnp.kron(I16, W3_root)
    b2t256 = jnp.tile(b2, 16)[None, :]
    b3t = jnp.tile(b3, 16)[None, :]

    zeros8 = jnp.zeros((TROWS, 8), f32)
    P1 = _sc_segsum(x8, srcp, dstp, zeros8, 8)
    P1p = P1.reshape(2, 6272, 128)
    x8p = x8.reshape(6272, 128)
    p2p, q2p = _tc1(P1p, x8p, BD1rel, BD1root, BD2rel, BD2root, b2t256)

    zeros16 = jnp.zeros((TROWS, 16), f32)
    p2 = p2p.reshape(NP, 16)
    P2 = _sc_segsum(p2, srcp, dstp, zeros16, 16)
    P2v = P2.reshape(2, 6272, 256)
    p3p, q3p = _tc2(P2v, q2p, BD3rel, BD3root, b2t256, b3t)

    p3 = p3p.reshape(NP, 8)
    P3 = _sc_segsum(p3, srcp, dstp, zeros8, 8)
    P3v = P3.reshape(2, 6272, 128)
    batchpad = jnp.pad(batch, (0, NP - N), constant_values=2 ** 30)
    batchp = jnp.repeat(batchpad, 8).reshape(6272, 128)
    pool128 = _tc3(P3v, q3p, batchp).reshape(G, 128)

    Wlin128 = jnp.pad(Wlin, ((0, 0), (0, 127)))
    blin128 = jnp.pad(blin[None, :], ((0, 0), (0, 127)))
    out128 = _tc4(pool128, Wlin128, blin128)
    return out128[:, 0]
